# baseline (speedup 1.0000x reference)
import functools

import jax
import jax.numpy as jnp
from jax import lax
from jax.experimental import pallas as pl
from jax.experimental.pallas import tpu as pltpu
from jax.experimental.pallas import tpu_sc as plsc

_NUM_TOKENS = 32768
_INPUT_SIZE = 768
_NUM_EXPERTS = 8
_BALANCE_LOSS_WEIGHT = 0.1
_BLOCK = 4096

_LANES = 16
_NUM_SUBCORES = 16
_HALF_TOKENS = _NUM_TOKENS // 2
_CHUNK = _HALF_TOKENS // _NUM_SUBCORES
_VECS = _CHUNK // _LANES


def _gate_kernel(x_ref, w1_ref, w2_ref,
                 idx_ref, score_ref, imp_ref):
    i = pl.program_id(0)
    nsteps = pl.num_programs(0)

    x = x_ref[...]
    w1 = w1_ref[...]
    w2 = w2_ref[...]

    ht = jnp.tanh(jax.lax.dot_general(
        w1, x, (((1,), (1,)), ((), ())),
        preferred_element_type=jnp.float32))
    logits = jax.lax.dot_general(
        w2, ht, (((1,), (0,)), ((), ())),
        preferred_element_type=jnp.float32)

    m = jnp.max(logits, axis=0, keepdims=True)
    iota = jax.lax.broadcasted_iota(jnp.int32, logits.shape, 0)
    idx = jnp.min(jnp.where(logits == m, iota, _NUM_EXPERTS), axis=0)
    idx_ref[...] = idx
    score_ref[...] = m[0]

    e = jnp.exp(logits - m)
    scores = e / jnp.sum(e, axis=0, keepdims=True)
    imp_part = jax.lax.dot_general(
        jnp.ones((1, scores.shape[1]), jnp.float32), scores,
        (((1,), (1,)), ((), ())),
        preferred_element_type=jnp.float32)
    imp16 = jnp.concatenate(
        [imp_part, jnp.zeros((1, _LANES - _NUM_EXPERTS), jnp.float32)],
        axis=1)[0]

    @pl.when(i == 0)
    def _init():
        imp_ref[...] = jnp.zeros_like(imp_ref)

    imp_ref[...] += imp16

    @pl.when(i == nsteps - 1)
    def _finalize():
        imp_ref[...] = imp_ref[...] * (1.0 / _NUM_TOKENS)


def _tc_gate_half(x, W1, W2, block_off):
    grid = (_HALF_TOKENS // _BLOCK,)
    return pl.pallas_call(
        _gate_kernel,
        grid=grid,
        in_specs=[
            pl.BlockSpec((_BLOCK, _INPUT_SIZE), lambda i: (i + block_off, 0)),
            pl.BlockSpec((_NUM_EXPERTS, _INPUT_SIZE), lambda i: (0, 0)),
            pl.BlockSpec((_NUM_EXPERTS, _NUM_EXPERTS), lambda i: (0, 0)),
        ],
        out_specs=[
            pl.BlockSpec((_BLOCK,), lambda i: (i,)),
            pl.BlockSpec((_BLOCK,), lambda i: (i,)),
            pl.BlockSpec((_LANES,), lambda i: (0,)),
        ],
        out_shape=[
            jax.ShapeDtypeStruct((_HALF_TOKENS,), jnp.int32),
            jax.ShapeDtypeStruct((_HALF_TOKENS,), jnp.float32),
            jax.ShapeDtypeStruct((_LANES,), jnp.float32),
        ],
    )(x, W1, W2)


def _count_into_bins(idx_hbm, idx_v, bins, sid):
    @pl.when(sid == 0)
    def _init_bins():
        for e in range(_NUM_EXPERTS):
            bins[e] = 0

    plsc.subcore_barrier()

    pltpu.sync_copy(idx_hbm.at[pl.ds(sid * _CHUNK, _CHUNK)], idx_v)

    def body(j, accs):
        v = idx_v[pl.ds(j * _LANES, _LANES)]
        return tuple(
            acc + plsc.all_reduce_population_count(v == e)
            for e, acc in enumerate(accs)
        )

    zero = jnp.zeros((_LANES,), jnp.int32)
    accs = lax.fori_loop(0, _VECS, body, (zero,) * _NUM_EXPERTS)

    for e in range(_NUM_EXPERTS):
        cnt_e = jnp.sum(accs[e], axis=0) // _LANES
        plsc.fetch_and_add(bins.at[e], cnt_e, subcore_id=0)

    plsc.subcore_barrier()


def _sc_count_body(idx_hbm, cnt_out, idx_v, out_v, bins):
    sid = lax.axis_index("s")
    lanes = lax.iota(jnp.int32, _LANES)
    _count_into_bins(idx_hbm, idx_v, bins, sid)

    @pl.when(sid == 0)
    def _emit():
        total = jnp.zeros((_LANES,), jnp.int32)
        for e in range(_NUM_EXPERTS):
            total = total + jnp.where(
                lanes == e, jnp.full((_LANES,), bins[e], jnp.int32), 0)
        out_v[...] = total
        pltpu.sync_copy(out_v.at[pl.ds(0, _NUM_EXPERTS)], cnt_out)


_sc_count = functools.partial(
    pl.kernel,
    mesh=plsc.VectorSubcoreMesh(
        core_axis_name="c", subcore_axis_name="s", num_cores=1),
    compiler_params=pltpu.CompilerParams(needs_layout_passes=False),
    out_type=[
        jax.ShapeDtypeStruct((_NUM_EXPERTS,), jnp.int32),
    ],
    scratch_types=[
        pltpu.VMEM((_CHUNK,), jnp.int32),
        pltpu.VMEM((_LANES,), jnp.int32),
        pltpu.SMEM((_NUM_EXPERTS,), jnp.int32),
    ],
)(_sc_count_body)


def _sc_final_body(idx_hbm, cnta_hbm, impa_hbm, impb_hbm,
                   load_out, loss_out, imp_out,
                   idx_v, cnta_v, imp_v, impb_v, out_v, bins):
    sid = lax.axis_index("s")
    lanes = lax.iota(jnp.int32, _LANES)
    _count_into_bins(idx_hbm, idx_v, bins, sid)

    @pl.when(sid == 0)
    def _finalize():
        cnta_v[...] = jnp.zeros((_LANES,), jnp.int32)
        pltpu.sync_copy(cnta_hbm, cnta_v.at[pl.ds(0, _NUM_EXPERTS)])
        total = cnta_v[...]
        for e in range(_NUM_EXPERTS):
            total = total + jnp.where(
                lanes == e, jnp.full((_LANES,), bins[e], jnp.int32), 0)
        load_mean = total.astype(jnp.float32) * (1.0 / _NUM_TOKENS)
        out_v[...] = load_mean
        pltpu.sync_copy(out_v.at[pl.ds(0, _NUM_EXPERTS)], load_out)

        pltpu.sync_copy(impa_hbm, imp_v)
        pltpu.sync_copy(impb_hbm, impb_v)
        imp = imp_v[...] + impb_v[...]
        imp_v[...] = imp
        pltpu.sync_copy(imp_v.at[pl.ds(0, _NUM_EXPERTS)], imp_out)
        prod = imp * load_mean
        loss = jnp.sum(prod, axis=0) * (_NUM_EXPERTS * _BALANCE_LOSS_WEIGHT)
        out_v[...] = jnp.full((_LANES,), loss, jnp.float32)
        pltpu.sync_copy(out_v.at[pl.ds(0, _NUM_EXPERTS)], loss_out)


_sc_final = functools.partial(
    pl.kernel,
    mesh=plsc.VectorSubcoreMesh(
        core_axis_name="c", subcore_axis_name="s", num_cores=1),
    compiler_params=pltpu.CompilerParams(needs_layout_passes=False),
    out_type=[
        jax.ShapeDtypeStruct((_NUM_EXPERTS,), jnp.float32),
        jax.ShapeDtypeStruct((_NUM_EXPERTS,), jnp.float32),
        jax.ShapeDtypeStruct((_NUM_EXPERTS,), jnp.float32),
    ],
    scratch_types=[
        pltpu.VMEM((_CHUNK,), jnp.int32),
        pltpu.VMEM((_LANES,), jnp.int32),
        pltpu.VMEM((_LANES,), jnp.float32),
        pltpu.VMEM((_LANES,), jnp.float32),
        pltpu.VMEM((_LANES,), jnp.float32),
        pltpu.SMEM((_NUM_EXPERTS,), jnp.int32),
    ],
)(_sc_final_body)


def kernel(x, W1, W2):
    n_half_blocks = _HALF_TOKENS // _BLOCK
    idx_a, score_a, imp_a = _tc_gate_half(x, W1, W2, 0)
    cnt_a, = _sc_count(idx_a)
    idx_b, score_b, imp_b = _tc_gate_half(x, W1, W2, n_half_blocks)
    load_mean, loss8, imp_mean = _sc_final(idx_b, cnt_a, imp_a, imp_b)
    idx = jnp.concatenate([idx_a, idx_b])
    score = jnp.concatenate([score_a, score_b])
    return (idx, score, loss8[0], load_mean, imp_mean)


# speedup vs baseline: 1.0860x; 1.0860x over previous
import functools

import jax
import jax.numpy as jnp
from jax import lax
from jax.experimental import pallas as pl
from jax.experimental.pallas import tpu as pltpu
from jax.experimental.pallas import tpu_sc as plsc

_NUM_TOKENS = 32768
_INPUT_SIZE = 768
_NUM_EXPERTS = 8
_BALANCE_LOSS_WEIGHT = 0.1
_BLOCK = 4096

_LANES = 16
_NUM_SUBCORES = 16
_CHUNK = _NUM_TOKENS // _NUM_SUBCORES
_VECS = _CHUNK // _LANES


def _gate_kernel(x_ref, w1_ref, w2_ref,
                 idx_ref, score_ref, imp_ref):
    i = pl.program_id(0)
    nsteps = pl.num_programs(0)

    x = x_ref[...]
    w1 = w1_ref[...]
    w2 = w2_ref[...]

    ht = jnp.tanh(jax.lax.dot_general(
        w1, x, (((1,), (1,)), ((), ())),
        preferred_element_type=jnp.float32))
    logits = jax.lax.dot_general(
        w2, ht, (((1,), (0,)), ((), ())),
        preferred_element_type=jnp.float32)

    m = jnp.max(logits, axis=0, keepdims=True)
    iota = jax.lax.broadcasted_iota(jnp.int32, logits.shape, 0)
    idx = jnp.min(jnp.where(logits == m, iota, _NUM_EXPERTS), axis=0)
    idx_ref[...] = idx
    score_ref[...] = m[0]

    e = jnp.exp(logits - m)
    scores = e / jnp.sum(e, axis=0, keepdims=True)
    imp_part = jax.lax.dot_general(
        jnp.ones((1, scores.shape[1]), jnp.float32), scores,
        (((1,), (1,)), ((), ())),
        preferred_element_type=jnp.float32)
    imp16 = jnp.concatenate(
        [imp_part, jnp.zeros((1, _LANES - _NUM_EXPERTS), jnp.float32)],
        axis=1)[0]

    @pl.when(i == 0)
    def _init():
        imp_ref[...] = jnp.zeros_like(imp_ref)

    imp_ref[...] += imp16

    @pl.when(i == nsteps - 1)
    def _finalize():
        imp_ref[...] = imp_ref[...] * (1.0 / _NUM_TOKENS)


def _tc_gate(x, W1, W2):
    n_tokens = x.shape[0]
    grid = (n_tokens // _BLOCK,)
    return pl.pallas_call(
        _gate_kernel,
        grid=grid,
        in_specs=[
            pl.BlockSpec((_BLOCK, _INPUT_SIZE), lambda i: (i, 0)),
            pl.BlockSpec((_NUM_EXPERTS, _INPUT_SIZE), lambda i: (0, 0)),
            pl.BlockSpec((_NUM_EXPERTS, _NUM_EXPERTS), lambda i: (0, 0)),
        ],
        out_specs=[
            pl.BlockSpec((_BLOCK,), lambda i: (i,)),
            pl.BlockSpec((_BLOCK,), lambda i: (i,)),
            pl.BlockSpec((_LANES,), lambda i: (0,)),
        ],
        out_shape=[
            jax.ShapeDtypeStruct((n_tokens,), jnp.int32),
            jax.ShapeDtypeStruct((n_tokens,), jnp.float32),
            jax.ShapeDtypeStruct((_LANES,), jnp.float32),
        ],
    )(x, W1, W2)


def _sc_body(idx_hbm, imp_hbm, load_out, loss_out, imp_out,
             idx_v, imp_v, out_v, bins):
    sid = lax.axis_index("s")
    lanes = lax.iota(jnp.int32, _LANES)

    @pl.when(sid == 0)
    def _init_bins():
        for e in range(_NUM_EXPERTS):
            bins[e] = 0

    plsc.subcore_barrier()

    pltpu.sync_copy(idx_hbm.at[pl.ds(sid * _CHUNK, _CHUNK)], idx_v)

    def body(j, accs):
        v = idx_v[pl.ds(j * _LANES, _LANES)]
        return tuple(
            acc + plsc.all_reduce_population_count(v == e)
            for e, acc in enumerate(accs)
        )

    zero = jnp.zeros((_LANES,), jnp.int32)
    accs = lax.fori_loop(0, _VECS, body, (zero,) * _NUM_EXPERTS)

    for e in range(_NUM_EXPERTS):
        cnt_e = jnp.sum(accs[e], axis=0) // _LANES
        plsc.fetch_and_add(bins.at[e], cnt_e, subcore_id=0)

    plsc.subcore_barrier()

    @pl.when(sid == 0)
    def _finalize():
        total = jnp.zeros((_LANES,), jnp.int32)
        for e in range(_NUM_EXPERTS):
            total = total + jnp.where(
                lanes == e, jnp.full((_LANES,), bins[e], jnp.int32), 0)
        load_mean = total.astype(jnp.float32) * (1.0 / _NUM_TOKENS)
        out_v[...] = load_mean
        pltpu.sync_copy(out_v.at[pl.ds(0, _NUM_EXPERTS)], load_out)

        pltpu.sync_copy(imp_hbm, imp_v)
        pltpu.sync_copy(imp_v.at[pl.ds(0, _NUM_EXPERTS)], imp_out)
        prod = imp_v[...] * load_mean
        loss = jnp.sum(prod, axis=0) * (_NUM_EXPERTS * _BALANCE_LOSS_WEIGHT)
        out_v[...] = jnp.full((_LANES,), loss, jnp.float32)
        pltpu.sync_copy(out_v.at[pl.ds(0, _NUM_EXPERTS)], loss_out)


_sc_router_stats = functools.partial(
    pl.kernel,
    mesh=plsc.VectorSubcoreMesh(
        core_axis_name="c", subcore_axis_name="s", num_cores=1),
    compiler_params=pltpu.CompilerParams(needs_layout_passes=False),
    out_type=[
        jax.ShapeDtypeStruct((_NUM_EXPERTS,), jnp.float32),
        jax.ShapeDtypeStruct((_NUM_EXPERTS,), jnp.float32),
        jax.ShapeDtypeStruct((_NUM_EXPERTS,), jnp.float32),
    ],
    scratch_types=[
        pltpu.VMEM((_CHUNK,), jnp.int32),
        pltpu.VMEM((_LANES,), jnp.float32),
        pltpu.VMEM((_LANES,), jnp.float32),
        pltpu.SMEM((_NUM_EXPERTS,), jnp.int32),
    ],
)(_sc_body)


def kernel(x, W1, W2):
    idx, score, imp16 = _tc_gate(x, W1, W2)
    load_mean, loss8, imp_mean = _sc_router_stats(idx, imp16)
    return (idx, score, loss8[0], load_mean, imp_mean)


# speedup vs baseline: 1.0921x; 1.0057x over previous
import functools

import jax
import jax.numpy as jnp
from jax import lax
from jax.experimental import pallas as pl
from jax.experimental.pallas import tpu as pltpu
from jax.experimental.pallas import tpu_sc as plsc

_NUM_TOKENS = 32768
_INPUT_SIZE = 768
_NUM_EXPERTS = 8
_BALANCE_LOSS_WEIGHT = 0.1
_BLOCK = 4096

_LANES = 16
_NUM_SUBCORES = 16
_CHUNK = _NUM_TOKENS // _NUM_SUBCORES
_VECS = _CHUNK // _LANES


def _gate_kernel(x_ref, w1_ref, w2_ref,
                 idx_ref, score_ref, imp_ref):
    i = pl.program_id(0)
    nsteps = pl.num_programs(0)

    x = x_ref[...]
    w1 = w1_ref[...]
    w2 = w2_ref[...]

    ht = jnp.tanh(jax.lax.dot_general(
        w1, x, (((1,), (1,)), ((), ())),
        preferred_element_type=jnp.float32))
    logits = jax.lax.dot_general(
        w2, ht, (((1,), (0,)), ((), ())),
        preferred_element_type=jnp.float32)

    m = jnp.max(logits, axis=0, keepdims=True)
    iota = jax.lax.broadcasted_iota(jnp.int32, logits.shape, 0)
    idx = jnp.min(jnp.where(logits == m, iota, _NUM_EXPERTS), axis=0)
    idx_ref[...] = idx
    score_ref[...] = m[0]

    e = jnp.exp(logits - m)
    scores = e / jnp.sum(e, axis=0, keepdims=True)
    imp_part = jax.lax.dot_general(
        jnp.ones((1, scores.shape[1]), jnp.float32), scores,
        (((1,), (1,)), ((), ())),
        preferred_element_type=jnp.float32)
    imp16 = jnp.concatenate(
        [imp_part, jnp.zeros((1, _LANES - _NUM_EXPERTS), jnp.float32)],
        axis=1)[0]

    @pl.when(i == 0)
    def _init():
        imp_ref[...] = jnp.zeros_like(imp_ref)

    imp_ref[...] += imp16

    @pl.when(i == nsteps - 1)
    def _finalize():
        imp_ref[...] = imp_ref[...] * (1.0 / _NUM_TOKENS)


def _tc_gate(x, W1, W2):
    n_tokens = x.shape[0]
    grid = (n_tokens // _BLOCK,)
    return pl.pallas_call(
        _gate_kernel,
        grid=grid,
        in_specs=[
            pl.BlockSpec((_BLOCK, _INPUT_SIZE), lambda i: (i, 0)),
            pl.BlockSpec((_NUM_EXPERTS, _INPUT_SIZE), lambda i: (0, 0)),
            pl.BlockSpec((_NUM_EXPERTS, _NUM_EXPERTS), lambda i: (0, 0)),
        ],
        out_specs=[
            pl.BlockSpec((_BLOCK,), lambda i: (i,)),
            pl.BlockSpec((_BLOCK,), lambda i: (i,)),
            pl.BlockSpec((_LANES,), lambda i: (0,)),
        ],
        out_shape=[
            jax.ShapeDtypeStruct((n_tokens,), jnp.int32),
            jax.ShapeDtypeStruct((n_tokens,), jnp.float32),
            jax.ShapeDtypeStruct((_LANES,), jnp.float32),
        ],
    )(x, W1, W2)


def _sc_body(idx_hbm, imp_hbm, load_out, loss_out, imp_out,
             idx_v, imp_v, out_v, bins):
    sid = lax.axis_index("s")
    lanes = lax.iota(jnp.int32, _LANES)

    @pl.when(sid == 0)
    def _init_bins():
        for e in range(_NUM_EXPERTS):
            bins[e] = 0

    plsc.subcore_barrier()

    pltpu.sync_copy(idx_hbm.at[pl.ds(sid * _CHUNK, _CHUNK)], idx_v)

    def body(j, accs):
        v = idx_v[pl.ds(j * _LANES, _LANES)]
        return tuple(
            acc + plsc.all_reduce_population_count(v == e)
            for e, acc in enumerate(accs)
        )

    zero = jnp.zeros((_LANES,), jnp.int32)
    accs = lax.fori_loop(0, _VECS, body, (zero,) * _NUM_EXPERTS)

    for e in range(_NUM_EXPERTS):
        cnt_e = jnp.sum(accs[e], axis=0) // _LANES
        plsc.fetch_and_add(bins.at[e], cnt_e, subcore_id=0)

    plsc.subcore_barrier()

    @pl.when(sid == 0)
    def _finalize():
        total = jnp.zeros((_LANES,), jnp.int32)
        for e in range(_NUM_EXPERTS):
            total = total + jnp.where(
                lanes == e, jnp.full((_LANES,), bins[e], jnp.int32), 0)
        load_mean = total.astype(jnp.float32) * (1.0 / _NUM_TOKENS)
        out_v[...] = load_mean
        pltpu.sync_copy(out_v.at[pl.ds(0, _NUM_EXPERTS)], load_out)

        pltpu.sync_copy(imp_hbm, imp_v)
        pltpu.sync_copy(imp_v.at[pl.ds(0, _NUM_EXPERTS)], imp_out)
        prod = imp_v[...] * load_mean
        loss = jnp.sum(prod, axis=0) * (_NUM_EXPERTS * _BALANCE_LOSS_WEIGHT)
        out_v[...] = jnp.full((_LANES,), loss, jnp.float32)
        pltpu.sync_copy(out_v.at[pl.ds(0, _NUM_EXPERTS)], loss_out)


def _sc_router_stats(idx, imp16):
    wrapped = functools.partial(
        pl.kernel,
        mesh=plsc.VectorSubcoreMesh(
            core_axis_name="c", subcore_axis_name="s", num_cores=1),
        compiler_params=pltpu.CompilerParams(needs_layout_passes=False),
        out_type=[
            jax.ShapeDtypeStruct((_NUM_EXPERTS,), jnp.float32),
            jax.ShapeDtypeStruct((_NUM_EXPERTS,), jnp.float32),
            jax.ShapeDtypeStruct((_NUM_EXPERTS,), jnp.float32),
        ],
        scratch_types=[
            pltpu.VMEM((_CHUNK,), jnp.int32),
            pltpu.VMEM((_LANES,), jnp.float32),
            pltpu.VMEM((_LANES,), jnp.float32),
            pltpu.SMEM((_NUM_EXPERTS,), jnp.int32),
        ],
    )(_sc_body)
    return wrapped(idx, imp16)


def kernel(x, W1, W2):
    idx, score, imp16 = _tc_gate(x, W1, W2)
    load_mean, loss8, imp_mean = _sc_router_stats(idx, imp16)
    return (idx, score, loss8[0], load_mean, imp_mean)
